# baseline (speedup 1.0000x reference)
import jax
import jax.numpy as jnp
from jax.experimental import pallas as pl


def _copy_body(src, dst):
    dst[...] = src[...]


def kernel(x, batch, pixel_batch, data_where, W_att, W_read):
    x_att = jax.nn.sigmoid(x @ W_att.T)
    _x = (x_att * x + x) / 2.0
    hg = jax.ops.segment_max(_x, batch, num_segments=64)
    logits = hg @ W_read.T
    gathered = x_att[pixel_batch]
    fv = jnp.zeros((64, 1, 64, 64), dtype=x_att.dtype)
    fv = fv.at[data_where[:, 0], :, data_where[:, 1], data_where[:, 2]].set(gathered)
    logits = pl.pallas_call(
        _copy_body, out_shape=jax.ShapeDtypeStruct(logits.shape, logits.dtype)
    )(logits)
    return (logits, fv)


# speedup vs baseline: 10.9992x; 10.9992x over previous
import functools

import jax
import jax.numpy as jnp
from jax import lax
from jax.experimental import pallas as pl
from jax.experimental.pallas import tpu as pltpu
from jax.experimental.pallas import tpu_sc as plsc

N = 50000
D = 128
P = 262144
B = 64
BN = 2000
NBLK = N // BN
NC = 2
NS = 16
NW = NC * NS
SLOTS = P // NW
WIN = 16384
NWIN = P // WIN
NPAD = 50048


def _dense_body(sf_ref, sl_ref, x_ref, b_ref, wa_ref, wrT_ref,
                xatt_ref, log_ref, hg_ref):
    i = pl.program_id(0)

    @pl.when(i == 0)
    def _():
        hg_ref[...] = jnp.full((B, D), -jnp.inf, jnp.float32)

    xb = x_ref[...]
    w = wa_ref[...]
    t = jnp.sum(xb * w, axis=1, keepdims=True)
    att = jax.nn.sigmoid(t)
    xatt_ref[...] = att
    scaled = xb * ((1.0 + att) * 0.5)

    bb = b_ref[...]
    s_first = sf_ref[i]
    s_last = sl_ref[i]

    def seg_body(s, _):
        mask = bb == s
        vals = jnp.max(jnp.where(mask, scaled, -jnp.inf), axis=0,
                       keepdims=True)
        cur = hg_ref[pl.ds(s, 1), :]
        hg_ref[pl.ds(s, 1), :] = jnp.maximum(cur, vals)
        return 0

    lax.fori_loop(s_first, s_last + 1, seg_body, 0)

    @pl.when(i == NBLK - 1)
    def _():
        log_ref[...] = jnp.dot(hg_ref[...], wrT_ref[...],
                               preferred_element_type=jnp.float32)


def _dense(x, batch, W_att, W_readT_pad, sfirst, slast):
    return pl.pallas_call(
        _dense_body,
        grid=(NBLK,),
        in_specs=[
            pl.BlockSpec(memory_space=pltpu.SMEM),
            pl.BlockSpec(memory_space=pltpu.SMEM),
            pl.BlockSpec((BN, D), lambda i: (i, 0)),
            pl.BlockSpec((BN, 1), lambda i: (i, 0)),
            pl.BlockSpec((1, D), lambda i: (0, 0)),
            pl.BlockSpec((D, 16), lambda i: (0, 0)),
        ],
        out_specs=[
            pl.BlockSpec((BN, 1), lambda i: (i, 0)),
            pl.BlockSpec((B, 16), lambda i: (0, 0)),
        ],
        out_shape=[
            jax.ShapeDtypeStruct((N, 1), jnp.float32),
            jax.ShapeDtypeStruct((B, 16), jnp.float32),
        ],
        scratch_shapes=[pltpu.VMEM((B, D), jnp.float32)],
    )(sfirst, slast, x, batch.reshape(N, 1), W_att, W_readT_pad)


def _flat_body(dw_ref, out_ref):
    r = dw_ref[...]
    out_ref[...] = r[0] * 4096 + r[1] * 64 + r[2]


def _flatten(dwt):
    out = pl.pallas_call(
        _flat_body,
        grid=(16,),
        in_specs=[pl.BlockSpec((3, 8, 2048), lambda i: (0, i, 0))],
        out_specs=pl.BlockSpec((8, 2048), lambda i: (i, 0)),
        out_shape=jax.ShapeDtypeStruct((128, 2048), jnp.int32),
    )(dwt.reshape(3, 128, 2048))
    return out.reshape(P)


def _sc_body(flat_hbm, pb_hbm, xatt_hbm, out_hbm,
             winner, idxbuf, wsafe, pbv, xattv, outv, sem):
    c = lax.axis_index("c")
    s = lax.axis_index("s")
    wid = c * NS + s
    base = pl.multiple_of(wid * SLOTS, SLOTS)
    lanes = lax.iota(jnp.int32, 16)

    pltpu.sync_copy(xatt_hbm, xattv)

    neg1 = jnp.full((16,), -1, jnp.int32)

    def init_body(i, _):
        winner[pl.ds(pl.multiple_of(i * 16, 16), 16)] = neg1
        return 0

    lax.fori_loop(0, SLOTS // 16, init_body, 0)

    for wwin in range(NWIN):
        pltpu.sync_copy(flat_hbm.at[pl.ds(wwin * WIN, WIN)], idxbuf)

        def scan_body(i, _, wbase=wwin * WIN):
            off = pl.multiple_of(i * 16, 16)
            idx = idxbuf[pl.ds(off, 16)]
            local = idx - base
            mask = plsc.bitcast(local, jnp.uint32) < jnp.uint32(SLOTS)
            localc = jnp.clip(local, 0, SLOTS - 1)
            pvec = lanes + (wbase + i * 16)
            cur = plsc.load_gather(winner, [localc], mask=mask)
            new = jnp.maximum(cur, pvec)
            plsc.store_scatter(winner, [localc], new, mask=mask)
            return 0

        lax.fori_loop(0, WIN // 16, scan_body, 0)

    def wsafe_body(i, _):
        off = pl.multiple_of(i * 16, 16)
        w_ = winner[pl.ds(off, 16)]
        dummy = lanes + (base + i * 16)
        wsafe[pl.ds(off, 16)] = jnp.where(w_ < 0, dummy, w_)
        return 0

    lax.fori_loop(0, SLOTS // 16, wsafe_body, 0)

    def gather_round(r, _):
        hs = []
        for jj in range(8):
            off = pl.multiple_of((r * 8 + jj) * 128, 128)
            hs.append(pltpu.make_async_copy(
                pb_hbm.at[wsafe.at[pl.ds(off, 128)]],
                pbv.at[pl.ds(off, 128)], sem))
        for h in hs:
            h.start()
        for h in hs:
            h.wait()
        return 0

    lax.fori_loop(0, SLOTS // (8 * 128), gather_round, 0)

    def out_body(i, _):
        off = pl.multiple_of(i * 16, 16)
        w_ = winner[pl.ds(off, 16)]
        pb_ = pbv[pl.ds(off, 16)]
        val = plsc.load_gather(xattv, [pb_])
        outv[pl.ds(off, 16)] = jnp.where(w_ >= 0, val, 0.0)
        return 0

    lax.fori_loop(0, SLOTS // 16, out_body, 0)

    pltpu.sync_copy(outv, out_hbm.at[pl.ds(base, SLOTS)])


def _sc_scatter(flat, pixel_batch, xatt_pad):
    mesh = plsc.VectorSubcoreMesh(core_axis_name="c", subcore_axis_name="s",
                                  num_cores=NC, num_subcores=NS)
    f = pl.kernel(
        _sc_body,
        out_type=jax.ShapeDtypeStruct((P,), jnp.float32),
        mesh=mesh,
        compiler_params=pltpu.CompilerParams(needs_layout_passes=False),
        scratch_types=[
            pltpu.VMEM((SLOTS,), jnp.int32),
            pltpu.VMEM((WIN,), jnp.int32),
            pltpu.VMEM((SLOTS,), jnp.int32),
            pltpu.VMEM((SLOTS,), jnp.int32),
            pltpu.VMEM((NPAD,), jnp.float32),
            pltpu.VMEM((SLOTS,), jnp.float32),
            pltpu.SemaphoreType.DMA,
        ],
    )
    return f(flat, pixel_batch, xatt_pad)


def kernel(x, batch, pixel_batch, data_where, W_att, W_read):
    batch = batch.astype(jnp.int32)
    pixel_batch = pixel_batch.astype(jnp.int32)
    data_where = data_where.astype(jnp.int32)

    sfirst = batch[0::BN]
    slast = batch[BN - 1::BN]
    W_readT_pad = jnp.pad(W_read, ((0, 6), (0, 0))).T

    x_att, logits_pad = _dense(x, batch, W_att, W_readT_pad, sfirst, slast)

    flat = _flatten(data_where.T)

    xatt_pad = jnp.pad(x_att.reshape(N), (0, NPAD - N))
    fv_flat = _sc_scatter(flat, pixel_batch, xatt_pad)

    return (logits_pad[:, :10], fv_flat.reshape(B, 1, 64, 64))


# speedup vs baseline: 11.2541x; 1.0232x over previous
import functools

import jax
import jax.numpy as jnp
from jax import lax
from jax.experimental import pallas as pl
from jax.experimental.pallas import tpu as pltpu
from jax.experimental.pallas import tpu_sc as plsc

N = 50000
D = 128
P = 262144
B = 64
BN = 2000
NBLK = N // BN
NC = 2
NS = 16
NW = NC * NS
SLOTS = P // NW
WIN = 16384
NWIN = P // WIN
NPAD = 50048


def _dense_body(sf_ref, sl_ref, x_ref, b_ref, wa_ref, wrT_ref,
                xatt_ref, log_ref, hg_ref):
    i = pl.program_id(0)

    @pl.when(i == 0)
    def _():
        hg_ref[...] = jnp.full((B, D), -jnp.inf, jnp.float32)

    xb = x_ref[...]
    w = wa_ref[...]
    t = jnp.sum(xb * w, axis=1, keepdims=True)
    att = jax.nn.sigmoid(t)
    xatt_ref[...] = att
    scaled = xb * ((1.0 + att) * 0.5)

    bb = b_ref[...]
    s_first = sf_ref[i]
    s_last = sl_ref[i]

    def seg_body(s, _):
        mask = bb == s
        vals = jnp.max(jnp.where(mask, scaled, -jnp.inf), axis=0,
                       keepdims=True)
        cur = hg_ref[pl.ds(s, 1), :]
        hg_ref[pl.ds(s, 1), :] = jnp.maximum(cur, vals)
        return 0

    lax.fori_loop(s_first, s_last + 1, seg_body, 0)

    @pl.when(i == NBLK - 1)
    def _():
        log_ref[...] = jnp.dot(hg_ref[...], wrT_ref[...],
                               preferred_element_type=jnp.float32)


def _dense(x, batch, W_att, W_readT_pad, sfirst, slast):
    return pl.pallas_call(
        _dense_body,
        grid=(NBLK,),
        in_specs=[
            pl.BlockSpec(memory_space=pltpu.SMEM),
            pl.BlockSpec(memory_space=pltpu.SMEM),
            pl.BlockSpec((BN, D), lambda i: (i, 0)),
            pl.BlockSpec((BN, 1), lambda i: (i, 0)),
            pl.BlockSpec((1, D), lambda i: (0, 0)),
            pl.BlockSpec((D, 16), lambda i: (0, 0)),
        ],
        out_specs=[
            pl.BlockSpec((BN, 1), lambda i: (i, 0)),
            pl.BlockSpec((B, 16), lambda i: (0, 0)),
        ],
        out_shape=[
            jax.ShapeDtypeStruct((N, 1), jnp.float32),
            jax.ShapeDtypeStruct((B, 16), jnp.float32),
        ],
        scratch_shapes=[pltpu.VMEM((B, D), jnp.float32)],
    )(sfirst, slast, x, batch.reshape(N, 1), W_att, W_readT_pad)


def _flat_body(dw_ref, out_ref):
    r = dw_ref[...]
    out_ref[...] = r[0] * 4096 + r[1] * 64 + r[2]


def _flatten(dwt):
    out = pl.pallas_call(
        _flat_body,
        grid=(16,),
        in_specs=[pl.BlockSpec((3, 8, 2048), lambda i: (0, i, 0))],
        out_specs=pl.BlockSpec((8, 2048), lambda i: (i, 0)),
        out_shape=jax.ShapeDtypeStruct((128, 2048), jnp.int32),
    )(dwt.reshape(3, 128, 2048))
    return out.reshape(P)


def _sc_body(flat_hbm, pb_hbm, xatt_hbm, out_hbm,
             winner, idxbuf, wsafe, pbv, xattv, outv, sem):
    c = lax.axis_index("c")
    s = lax.axis_index("s")
    wid = c * NS + s
    base = pl.multiple_of(wid * SLOTS, SLOTS)
    lanes = lax.iota(jnp.int32, 16)

    pltpu.sync_copy(xatt_hbm, xattv)

    neg1 = jnp.full((16,), -1, jnp.int32)

    def init_body(i, _):
        winner[pl.ds(pl.multiple_of(i * 16, 16), 16)] = neg1
        return 0

    lax.fori_loop(0, SLOTS // 16, init_body, 0, unroll=8)

    for wwin in range(NWIN):
        pltpu.sync_copy(flat_hbm.at[pl.ds(wwin * WIN, WIN)], idxbuf)

        def scan_body(i, _, wbase=wwin * WIN):
            off = pl.multiple_of(i * 16, 16)
            idx = idxbuf[pl.ds(off, 16)]
            local = idx - base
            mask = plsc.bitcast(local, jnp.uint32) < jnp.uint32(SLOTS)
            localc = jnp.clip(local, 0, SLOTS - 1)
            pvec = lanes + (wbase + i * 16)
            cur = plsc.load_gather(winner, [localc], mask=mask)
            new = jnp.maximum(cur, pvec)
            plsc.store_scatter(winner, [localc], new, mask=mask)
            return 0

        lax.fori_loop(0, WIN // 16, scan_body, 0, unroll=8)

    def wsafe_body(i, _):
        off = pl.multiple_of(i * 16, 16)
        w_ = winner[pl.ds(off, 16)]
        dummy = lanes + (base + i * 16)
        wsafe[pl.ds(off, 16)] = jnp.where(w_ < 0, dummy, w_)
        return 0

    lax.fori_loop(0, SLOTS // 16, wsafe_body, 0, unroll=8)

    def gather_round(r, _):
        hs = []
        for jj in range(8):
            off = pl.multiple_of((r * 8 + jj) * 128, 128)
            hs.append(pltpu.make_async_copy(
                pb_hbm.at[wsafe.at[pl.ds(off, 128)]],
                pbv.at[pl.ds(off, 128)], sem))
        for h in hs:
            h.start()
        for h in hs:
            h.wait()
        return 0

    lax.fori_loop(0, SLOTS // (8 * 128), gather_round, 0)

    def out_body(i, _):
        off = pl.multiple_of(i * 16, 16)
        w_ = winner[pl.ds(off, 16)]
        pb_ = pbv[pl.ds(off, 16)]
        val = plsc.load_gather(xattv, [pb_])
        outv[pl.ds(off, 16)] = jnp.where(w_ >= 0, val, 0.0)
        return 0

    lax.fori_loop(0, SLOTS // 16, out_body, 0, unroll=8)

    pltpu.sync_copy(outv, out_hbm.at[pl.ds(base, SLOTS)])


def _sc_scatter(flat, pixel_batch, xatt_pad):
    mesh = plsc.VectorSubcoreMesh(core_axis_name="c", subcore_axis_name="s",
                                  num_cores=NC, num_subcores=NS)
    f = pl.kernel(
        _sc_body,
        out_type=jax.ShapeDtypeStruct((P,), jnp.float32),
        mesh=mesh,
        compiler_params=pltpu.CompilerParams(needs_layout_passes=False),
        scratch_types=[
            pltpu.VMEM((SLOTS,), jnp.int32),
            pltpu.VMEM((WIN,), jnp.int32),
            pltpu.VMEM((SLOTS,), jnp.int32),
            pltpu.VMEM((SLOTS,), jnp.int32),
            pltpu.VMEM((NPAD,), jnp.float32),
            pltpu.VMEM((SLOTS,), jnp.float32),
            pltpu.SemaphoreType.DMA,
        ],
    )
    return f(flat, pixel_batch, xatt_pad)


def kernel(x, batch, pixel_batch, data_where, W_att, W_read):
    batch = batch.astype(jnp.int32)
    pixel_batch = pixel_batch.astype(jnp.int32)
    data_where = data_where.astype(jnp.int32)

    sfirst = batch[0::BN]
    slast = batch[BN - 1::BN]
    W_readT_pad = jnp.pad(W_read, ((0, 6), (0, 0))).T

    x_att, logits_pad = _dense(x, batch, W_att, W_readT_pad, sfirst, slast)

    flat = _flatten(data_where.T)

    xatt_pad = jnp.pad(x_att.reshape(N), (0, NPAD - N))
    fv_flat = _sc_scatter(flat, pixel_batch, xatt_pad)

    return (logits_pad[:, :10], fv_flat.reshape(B, 1, 64, 64))


# speedup vs baseline: 23.0607x; 2.0491x over previous
import functools

import jax
import jax.numpy as jnp
from jax import lax
from jax.experimental import pallas as pl
from jax.experimental.pallas import tpu as pltpu
from jax.experimental.pallas import tpu_sc as plsc

N = 50000
D = 128
P = 262144
B = 64
BN = 2000
NBLK = N // BN
NC = 2
NS = 16
NW = NC * NS
SLOTS = P // NW
WIN = 16384
NWIN = P // WIN
NPAD = 50048


def _dense_body(sf_ref, sl_ref, x_ref, b_ref, wa_ref, wrT_ref,
                xatt_ref, log_ref, hg_ref):
    i = pl.program_id(0)

    @pl.when(i == 0)
    def _():
        hg_ref[...] = jnp.full((B, D), -jnp.inf, jnp.float32)

    xb = x_ref[...]
    w = wa_ref[...]
    t = jnp.sum(xb * w, axis=1, keepdims=True)
    att = jax.nn.sigmoid(t)
    xatt_ref[...] = att
    scaled = xb * ((1.0 + att) * 0.5)

    bb = b_ref[...]
    s_first = sf_ref[i]
    s_last = sl_ref[i]

    def seg_body(s, _):
        mask = bb == s
        vals = jnp.max(jnp.where(mask, scaled, -jnp.inf), axis=0,
                       keepdims=True)
        cur = hg_ref[pl.ds(s, 1), :]
        hg_ref[pl.ds(s, 1), :] = jnp.maximum(cur, vals)
        return 0

    lax.fori_loop(s_first, s_last + 1, seg_body, 0)

    @pl.when(i == NBLK - 1)
    def _():
        log_ref[...] = jnp.dot(hg_ref[...], wrT_ref[...],
                               preferred_element_type=jnp.float32)


def _dense(x, batch, W_att, W_readT_pad, sfirst, slast):
    return pl.pallas_call(
        _dense_body,
        grid=(NBLK,),
        in_specs=[
            pl.BlockSpec(memory_space=pltpu.SMEM),
            pl.BlockSpec(memory_space=pltpu.SMEM),
            pl.BlockSpec((BN, D), lambda i: (i, 0)),
            pl.BlockSpec((BN, 1), lambda i: (i, 0)),
            pl.BlockSpec((1, D), lambda i: (0, 0)),
            pl.BlockSpec((D, 16), lambda i: (0, 0)),
        ],
        out_specs=[
            pl.BlockSpec((BN, 1), lambda i: (i, 0)),
            pl.BlockSpec((B, 16), lambda i: (0, 0)),
        ],
        out_shape=[
            jax.ShapeDtypeStruct((N, 1), jnp.float32),
            jax.ShapeDtypeStruct((B, 16), jnp.float32),
        ],
        scratch_shapes=[pltpu.VMEM((B, D), jnp.float32)],
    )(sfirst, slast, x, batch.reshape(N, 1), W_att, W_readT_pad)


def _flat_body(dw_ref, out_ref):
    r = dw_ref[...]
    out_ref[...] = r[0] * 4096 + r[1] * 64 + r[2]


def _flatten(dwt):
    out = pl.pallas_call(
        _flat_body,
        grid=(16,),
        in_specs=[pl.BlockSpec((3, 8, 2048), lambda i: (0, i, 0))],
        out_specs=pl.BlockSpec((8, 2048), lambda i: (i, 0)),
        out_shape=jax.ShapeDtypeStruct((128, 2048), jnp.int32),
    )(dwt.reshape(3, 128, 2048))
    return out.reshape(P)


def _sc_body(flat_hbm, pb_hbm, xatt_hbm, out_hbm,
             vals, idxbuf, pbbuf, xattv, semx, sia, sib, spa, spb):
    c = lax.axis_index("c")
    s = lax.axis_index("s")
    wid = c * NS + s
    base = pl.multiple_of(wid * SLOTS, SLOTS)

    xcp = pltpu.make_async_copy(xatt_hbm, xattv, semx)
    xcp.start()

    zero16 = jnp.zeros((16,), jnp.float32)

    def init_body(i, _):
        vals[pl.ds(pl.multiple_of(i * 16, 16), 16)] = zero16
        return 0

    lax.fori_loop(0, SLOTS // 16, init_body, 0, unroll=8)

    def win_copies(w):
        buf = (w % 2) * WIN
        si = sia if w % 2 == 0 else sib
        sp = spa if w % 2 == 0 else spb
        return (
            pltpu.make_async_copy(flat_hbm.at[pl.ds(w * WIN, WIN)],
                                  idxbuf.at[pl.ds(buf, WIN)], si),
            pltpu.make_async_copy(pb_hbm.at[pl.ds(w * WIN, WIN)],
                                  pbbuf.at[pl.ds(buf, WIN)], sp),
        )

    h = win_copies(0)
    for x in h:
        x.start()
    xcp.wait()

    for w in range(NWIN):
        for x in h:
            x.wait()
        if w + 1 < NWIN:
            h = win_copies(w + 1)
            for x in h:
                x.start()
        buf = (w % 2) * WIN

        def scan_body(i, _, buf=buf):
            offs = [pl.multiple_of(buf + (i * 8 + k) * 16, 16)
                    for k in range(8)]
            idxs = [idxbuf[pl.ds(o, 16)] for o in offs]
            pbs = [pbbuf[pl.ds(o, 16)] for o in offs]
            locs = [idx - base for idx in idxs]
            masks = [plsc.bitcast(l, jnp.uint32) < jnp.uint32(SLOTS)
                     for l in locs]
            vs = [plsc.load_gather(xattv, [pb_]) for pb_ in pbs]
            for l, v, m in zip(locs, vs, masks):
                plsc.store_scatter(vals, [l], v, mask=m)
            return 0

        lax.fori_loop(0, WIN // 128, scan_body, 0, unroll=2)

    pltpu.sync_copy(vals, out_hbm.at[pl.ds(base, SLOTS)])


def _sc_scatter(flat, pixel_batch, xatt_pad):
    mesh = plsc.VectorSubcoreMesh(core_axis_name="c", subcore_axis_name="s",
                                  num_cores=NC, num_subcores=NS)
    f = pl.kernel(
        _sc_body,
        out_type=jax.ShapeDtypeStruct((P,), jnp.float32),
        mesh=mesh,
        compiler_params=pltpu.CompilerParams(needs_layout_passes=False),
        scratch_types=[
            pltpu.VMEM((SLOTS,), jnp.float32),
            pltpu.VMEM((2 * WIN,), jnp.int32),
            pltpu.VMEM((2 * WIN,), jnp.int32),
            pltpu.VMEM((NPAD,), jnp.float32),
            pltpu.SemaphoreType.DMA,
            pltpu.SemaphoreType.DMA,
            pltpu.SemaphoreType.DMA,
            pltpu.SemaphoreType.DMA,
            pltpu.SemaphoreType.DMA,
        ],
    )
    return f(flat, pixel_batch, xatt_pad)


def kernel(x, batch, pixel_batch, data_where, W_att, W_read):
    batch = batch.astype(jnp.int32)
    pixel_batch = pixel_batch.astype(jnp.int32)
    data_where = data_where.astype(jnp.int32)

    sfirst = batch[0::BN]
    slast = batch[BN - 1::BN]
    W_readT_pad = jnp.pad(W_read, ((0, 6), (0, 0))).T

    x_att, logits_pad = _dense(x, batch, W_att, W_readT_pad, sfirst, slast)

    flat = _flatten(data_where.T)

    xatt_pad = jnp.pad(x_att.reshape(N), (0, NPAD - N))
    fv_flat = _sc_scatter(flat, pixel_batch, xatt_pad)

    return (logits_pad[:, :10], fv_flat.reshape(B, 1, 64, 64))
